# baseline (speedup 1.0000x reference)
import functools

import jax
import jax.numpy as jnp
from jax import lax
from jax.experimental import pallas as pl
from jax.experimental.pallas import tpu as pltpu
from jax.experimental.pallas import tpu_sc as plsc

E = 64
IN = 768
HID = 1536
OUT = 768
NB = 8
R = 8
ALPHA = 16.0
K = 2
N = 2048
SCALING = ALPHA / R

BT = 128
MAXB = N * K // BT + E
P = MAXB * BT

NEG = -3.0e38

NC = 2
NS = 16
NW = NC * NS


def _gating_kernel(x_ref, wg_ref, xp_ref, d0_ref, d1_ref, g1_ref, g2_ref,
                   be_ref, nb_ref, loss_ref):
    x = x_ref[...]

    au = lax.bitcast_convert_type(
        x[:, :_PW].astype(jnp.bfloat16), jnp.uint16).astype(jnp.uint32)
    bu = lax.bitcast_convert_type(
        x[:, _PW:].astype(jnp.bfloat16), jnp.uint16).astype(jnp.uint32)
    xp_ref[...] = lax.bitcast_convert_type(au | (bu << 16), jnp.float32)
    logits = jnp.dot(x, wg_ref[...], preferred_element_type=jnp.float32)
    iota = lax.broadcasted_iota(jnp.int32, (N, E), 1)
    m1 = jnp.max(logits, axis=1, keepdims=True)
    idx1 = jnp.min(jnp.where(logits == m1, iota, E), axis=1, keepdims=True)
    sel1 = iota == idx1
    l2 = jnp.where(sel1, NEG, logits)
    m2 = jnp.max(l2, axis=1, keepdims=True)
    idx2 = jnp.min(jnp.where(l2 == m2, iota, E), axis=1, keepdims=True)
    sel2 = iota == idx2
    ed = jnp.exp(m2 - m1)
    g1 = 1.0 / (1.0 + ed)
    g2 = ed / (1.0 + ed)

    g1_ref[...] = g1
    g2_ref[...] = g2

    oh1 = sel1.astype(jnp.float32)
    oh2 = sel2.astype(jnp.float32)

    ri = lax.broadcasted_iota(jnp.int32, (N, N), 0)
    ci = lax.broadcasted_iota(jnp.int32, (N, N), 1)
    lt = (ci < ri).astype(jnp.float32)
    oh = jnp.concatenate([oh1, oh2], axis=1)
    prefix = jnp.dot(lt, oh, preferred_element_type=jnp.float32)
    p1 = prefix[:, :E]
    p2 = prefix[:, E:]
    c1 = jnp.sum(oh1, axis=0, keepdims=True)
    rank0 = jnp.sum(jnp.where(sel1, p1, 0.0), axis=1, keepdims=True)
    rank1 = jnp.sum(jnp.where(sel2, c1 + p2, 0.0), axis=1, keepdims=True)
    counts = c1 + jnp.sum(oh2, axis=0, keepdims=True)

    nb = jnp.floor((counts + (BT - 1)) * (1.0 / BT))
    ei = lax.broadcasted_iota(jnp.int32, (E, E), 0)
    ej = lax.broadcasted_iota(jnp.int32, (E, E), 1)
    ends = jnp.dot(nb, (ei <= ej).astype(jnp.float32),
                   preferred_element_type=jnp.float32)
    pad_off = (ends - nb) * BT

    d0 = jnp.sum(jnp.where(sel1, pad_off, 0.0), axis=1, keepdims=True) + rank0
    d1 = jnp.sum(jnp.where(sel2, pad_off, 0.0), axis=1, keepdims=True) + rank1
    d0_ref[...] = d0.astype(jnp.int32)
    d1_ref[...] = d1.astype(jnp.int32)

    bj = lax.broadcasted_iota(jnp.int32, (MAXB, 1), 0).astype(jnp.float32)
    total = jnp.sum(nb)
    be_raw = jnp.sum((jnp.broadcast_to(ends, (MAXB, E)) <= bj)
                     .astype(jnp.float32), axis=1, keepdims=True)
    be_last = jnp.sum(jnp.where(bj == total - 1.0, be_raw, 0.0))
    be = jnp.where(bj < total, be_raw, be_last)
    be_ref[...] = be.astype(jnp.int32)
    nb_ref[0, 0] = total.astype(jnp.int32)

    gates = jnp.where(sel1, g1, 0.0) + jnp.where(sel2, g2, 0.0)
    importance = jnp.sum(gates, axis=0)
    load = jnp.sum((gates > 0).astype(jnp.float32), axis=0)

    def cv_sq(v):
        mean = jnp.mean(v)
        var = jnp.sum((v - mean) ** 2) / (E - 1)
        return var / (mean * mean + 1e-10)

    loss_ref[0, 0] = (cv_sq(importance) + cv_sq(load)) * 0.01


def _gmm_kernel(be_ref, nb_ref, xd_ref, bv_ref, gv_ref,
                w1_ref, b1_ref, w2_ref, b2_ref,
                a1_ref, bb1_ref, a2_ref, bb2_ref, out_ref):
    @pl.when(pl.program_id(0) < nb_ref[0])
    def _():
        u = lax.bitcast_convert_type(xd_ref[...], jnp.uint32)
        xa = lax.bitcast_convert_type(u << 16, jnp.float32)
        xb = lax.bitcast_convert_type((u >> 16) << 16, jnp.float32)
        x = jnp.concatenate([xa, xb], axis=1)
        bands = bv_ref[0]
        iota_nbr = lax.broadcasted_iota(jnp.int32, (BT, NB * R), 1)
        mask = (lax.div(iota_nbr, R) == bands).astype(jnp.float32)

        lh = jnp.dot(x, a1_ref[0], preferred_element_type=jnp.float32) * mask
        lh = jnp.dot(lh, bb1_ref[0], preferred_element_type=jnp.float32)
        h = jnp.dot(x, w1_ref[0], preferred_element_type=jnp.float32)
        h = h + b1_ref[0] + lh * SCALING
        h = h * 0.5 * (1.0 + lax.erf(h * 0.7071067811865476))

        lo = jnp.dot(h, a2_ref[0], preferred_element_type=jnp.float32) * mask
        lo = jnp.dot(lo, bb2_ref[0], preferred_element_type=jnp.float32)
        out = jnp.dot(h, w2_ref[0], preferred_element_type=jnp.float32)
        out = out + b2_ref[0] + lo * SCALING
        out_ref[...] = out * gv_ref[0]


_PW = IN // 2
_TPW = N // NW


@functools.lru_cache(maxsize=None)
def _build_sc_dispatch():
    @functools.partial(
        pl.kernel,
        mesh=plsc.VectorSubcoreMesh(core_axis_name="c", subcore_axis_name="s"),
        out_type=(
            jax.ShapeDtypeStruct((P, _PW), jnp.float32),
            jax.ShapeDtypeStruct((P,), jnp.float32),
            jax.ShapeDtypeStruct((P,), jnp.int32),
        ),
        scratch_types=[
            pltpu.VMEM((_TPW,), jnp.int32),
            pltpu.VMEM((_TPW,), jnp.int32),
            pltpu.VMEM((_TPW, _PW), jnp.float32),
            pltpu.VMEM((_TPW,), jnp.float32),
            pltpu.VMEM((_TPW,), jnp.float32),
            pltpu.VMEM((_TPW,), jnp.int32),
        ] + [pltpu.SemaphoreType.DMA] * 6,
    )
    def k(x_hbm, d0_hbm, d1_hbm, g1_hbm, g2_hbm, bd_hbm,
          xd_hbm, gv_hbm, bv_hbm,
          i0_v, i1_v, rows_v, g1_v, g2_v, bd_v, *sems):
        wid = lax.axis_index("s") * NC + lax.axis_index("c")
        base = wid * _TPW
        pltpu.sync_copy(d0_hbm.at[pl.ds(base, _TPW)], i0_v)
        pltpu.sync_copy(d1_hbm.at[pl.ds(base, _TPW)], i1_v)
        pltpu.sync_copy(x_hbm.at[pl.ds(base, _TPW)], rows_v)
        pltpu.sync_copy(g1_hbm.at[pl.ds(base, _TPW)], g1_v)
        pltpu.sync_copy(g2_hbm.at[pl.ds(base, _TPW)], g2_v)
        pltpu.sync_copy(bd_hbm.at[pl.ds(base, _TPW)], bd_v)
        copies = [
            pltpu.async_copy(rows_v, xd_hbm.at[i0_v], sems[0]),
            pltpu.async_copy(rows_v, xd_hbm.at[i1_v], sems[1]),
            pltpu.async_copy(g1_v, gv_hbm.at[i0_v], sems[2]),
            pltpu.async_copy(g2_v, gv_hbm.at[i1_v], sems[3]),
            pltpu.async_copy(bd_v, bv_hbm.at[i0_v], sems[4]),
            pltpu.async_copy(bd_v, bv_hbm.at[i1_v], sems[5]),
        ]
        for c in copies:
            c.wait()
    return k


@functools.lru_cache(maxsize=None)
def _build_sc_combine():
    @functools.partial(
        pl.kernel,
        mesh=plsc.VectorSubcoreMesh(core_axis_name="c", subcore_axis_name="s"),
        out_type=jax.ShapeDtypeStruct((N, OUT), jnp.float32),
        scratch_types=[
            pltpu.VMEM((_TPW,), jnp.int32),
            pltpu.VMEM((_TPW,), jnp.int32),
            pltpu.VMEM((_TPW, OUT), jnp.float32),
            pltpu.VMEM((_TPW, OUT), jnp.float32),
            pltpu.SemaphoreType.DMA,
        ],
    )
    def k(outw_hbm, d0_hbm, d1_hbm, y_hbm, i0_v, i1_v, r0_v, r1_v, sem):
        wid = lax.axis_index("s") * NC + lax.axis_index("c")
        base = wid * _TPW
        pltpu.sync_copy(d0_hbm.at[pl.ds(base, _TPW)], i0_v)
        pltpu.sync_copy(d1_hbm.at[pl.ds(base, _TPW)], i1_v)
        pltpu.async_copy(outw_hbm.at[i0_v], r0_v, sem).wait()
        pltpu.async_copy(outw_hbm.at[i1_v], r1_v, sem).wait()

        def body(t, _):
            def cbody(j, _):
                cs = pl.ds(j * 16, 16)
                r0_v[t, cs] = r0_v[t, cs] + r1_v[t, cs]
                return 0
            return lax.fori_loop(0, OUT // 16, cbody, 0)

        lax.fori_loop(0, _TPW, body, 0)
        pltpu.sync_copy(r0_v, y_hbm.at[pl.ds(base, _TPW)])
    return k


def _sc_dispatch(x, dest0, dest1, g1, g2, bands):
    return _build_sc_dispatch()(x, dest0, dest1, g1, g2, bands)


def _sc_combine(outw, dest0, dest1):
    return _build_sc_combine()(outw, dest0, dest1)


def kernel(x, band_indices, w_gate, fc1_W, fc1_b, fc2_W, fc2_b,
           lora1_A, lora1_B, lora2_A, lora2_B):
    x_pack, d0, d1, g1, g2, be, nb_tot, loss = pl.pallas_call(
        _gating_kernel,
        out_shape=(
            jax.ShapeDtypeStruct((N, _PW), jnp.float32),
            jax.ShapeDtypeStruct((N, 1), jnp.int32),
            jax.ShapeDtypeStruct((N, 1), jnp.int32),
            jax.ShapeDtypeStruct((N, 1), jnp.float32),
            jax.ShapeDtypeStruct((N, 1), jnp.float32),
            jax.ShapeDtypeStruct((MAXB, 1), jnp.int32),
            jax.ShapeDtypeStruct((1, 1), jnp.int32),
            jax.ShapeDtypeStruct((1, 1), jnp.float32),
        ),
        in_specs=[
            pl.BlockSpec((N, IN), lambda: (0, 0)),
            pl.BlockSpec((IN, E), lambda: (0, 0)),
        ],
        out_specs=(
            pl.BlockSpec((N, _PW), lambda: (0, 0)),
            pl.BlockSpec((N, 1), lambda: (0, 0)),
            pl.BlockSpec((N, 1), lambda: (0, 0)),
            pl.BlockSpec((N, 1), lambda: (0, 0)),
            pl.BlockSpec((N, 1), lambda: (0, 0)),
            pl.BlockSpec((MAXB, 1), lambda: (0, 0)),
            pl.BlockSpec(memory_space=pltpu.SMEM),
            pl.BlockSpec(memory_space=pltpu.SMEM),
        ),
    )(x, w_gate)

    dest0 = d0.reshape(N)
    dest1 = d1.reshape(N)
    block_expert = be.reshape(MAXB)
    total_blocks = nb_tot.reshape(1)
    bands = band_indices.astype(jnp.int32)

    xd, gv, bv = _sc_dispatch(x_pack, dest0, dest1,
                              g1.reshape(N), g2.reshape(N), bands)

    a1f = lora1_A.transpose(0, 2, 1, 3).reshape(E, IN, NB * R)
    bb1f = lora1_B.reshape(E, NB * R, HID)
    a2f = lora2_A.transpose(0, 2, 1, 3).reshape(E, HID, NB * R)
    bb2f = lora2_B.reshape(E, NB * R, OUT)
    b1_3d = fc1_b.reshape(E, 1, HID)
    b2_3d = fc2_b.reshape(E, 1, OUT)
    bv3 = bv.reshape(MAXB, BT, 1)
    gv3 = gv.reshape(MAXB, BT, 1)

    grid_spec = pltpu.PrefetchScalarGridSpec(
        num_scalar_prefetch=2,
        grid=(MAXB,),
        in_specs=[
            pl.BlockSpec((BT, _PW), lambda i, be, nb: (i, 0)),
            pl.BlockSpec((1, BT, 1), lambda i, be, nb: (i, 0, 0)),
            pl.BlockSpec((1, BT, 1), lambda i, be, nb: (i, 0, 0)),
            pl.BlockSpec((1, IN, HID), lambda i, be, nb: (be[i], 0, 0)),
            pl.BlockSpec((1, 1, HID), lambda i, be, nb: (be[i], 0, 0)),
            pl.BlockSpec((1, HID, OUT), lambda i, be, nb: (be[i], 0, 0)),
            pl.BlockSpec((1, 1, OUT), lambda i, be, nb: (be[i], 0, 0)),
            pl.BlockSpec((1, IN, NB * R), lambda i, be, nb: (be[i], 0, 0)),
            pl.BlockSpec((1, NB * R, HID), lambda i, be, nb: (be[i], 0, 0)),
            pl.BlockSpec((1, HID, NB * R), lambda i, be, nb: (be[i], 0, 0)),
            pl.BlockSpec((1, NB * R, OUT), lambda i, be, nb: (be[i], 0, 0)),
        ],
        out_specs=pl.BlockSpec((BT, OUT), lambda i, be, nb: (i, 0)),
    )
    outw = pl.pallas_call(
        _gmm_kernel,
        grid_spec=grid_spec,
        out_shape=jax.ShapeDtypeStruct((P, OUT), jnp.float32),
    )(block_expert, total_blocks, xd, bv3, gv3, fc1_W, b1_3d,
      fc2_W, b2_3d, a1f, bb1f, a2f, bb2f)

    y = _sc_combine(outw, dest0, dest1)

    return y, loss[0, 0]
